# Initial kernel scaffold; baseline (speedup 1.0000x reference)
import jax
import jax.numpy as jnp
from jax.experimental import pallas as pl


def kernel(x, edge_index, W1, b1, W2, b2):
    raise NotImplementedError("write your pallas kernel here")


# speedup vs baseline: 9.3228x; 9.3228x over previous
import jax
import jax.numpy as jnp
from jax import lax
from jax.experimental import pallas as pl
from jax.experimental.pallas import tpu as pltpu
from jax.experimental.pallas import tpu_sc as plsc

N, E, D, C = 10000, 320000, 128, 128
NC, NS = 2, 16
DH = D // NC
EPW = E // NS
K = 125
NCH = EPW // K
CPC = NCH // NC
NPAD = 10240
RPT = NPAD // NS


def _sc_body(xl_hbm, xr_hbm, src_hbm, dst_hbm, z64_hbm, z16_hbm, o16_hbm,
             agg_out, cnt_out,
             src_v, dst_v, rows_v, ones_v, acc_sh, cnt_sh, sem):
    cid = lax.axis_index("c")
    sid = lax.axis_index("s")

    row0 = sid * RPT
    pltpu.sync_copy(z64_hbm, acc_sh.at[pl.ds(row0, RPT)])
    pltpu.sync_copy(z16_hbm, cnt_sh.at[pl.ds(row0, RPT)])

    pltpu.sync_copy(src_hbm.at[sid], src_v)
    pltpu.sync_copy(dst_hbm.at[sid], dst_v)
    pltpu.sync_copy(o16_hbm, ones_v)

    plsc.subcore_barrier()

    def chunk(j, carry):
        @pl.when(cid == 0)
        def _():
            pltpu.async_copy(xl_hbm.at[src_v.at[j]], rows_v, sem).wait()

        @pl.when(cid == 1)
        def _():
            pltpu.async_copy(xr_hbm.at[src_v.at[j]], rows_v, sem).wait()

        pltpu.sync_copy(rows_v, acc_sh.at[dst_v.at[j]], add=True)

        @pl.when((j >= cid * CPC) & (j < (cid + 1) * CPC))
        def _():
            pltpu.sync_copy(ones_v, cnt_sh.at[src_v.at[j]], add=True)

        return carry

    lax.fori_loop(0, NCH, chunk, 0)

    plsc.subcore_barrier()

    pltpu.sync_copy(acc_sh.at[pl.ds(row0, RPT)],
                    agg_out.at[cid, pl.ds(row0, RPT)])
    pltpu.sync_copy(cnt_sh.at[pl.ds(row0, RPT)],
                    cnt_out.at[cid, pl.ds(row0, RPT)])


@jax.jit
def _sc_scatter(xl, xr, src3, dst3, z64, z16, o16):
    mesh = plsc.VectorSubcoreMesh(core_axis_name="c", subcore_axis_name="s")
    return pl.kernel(
        _sc_body,
        out_type=(jax.ShapeDtypeStruct((NC, NPAD, DH), jnp.float32),
                  jax.ShapeDtypeStruct((NC, NPAD, 16), jnp.float32)),
        mesh=mesh,
        scratch_types=[
            pltpu.VMEM((NCH, K), jnp.int32),
            pltpu.VMEM((NCH, K), jnp.int32),
            pltpu.VMEM((K, DH), jnp.float32),
            pltpu.VMEM((K, 16), jnp.float32),
            pltpu.VMEM_SHARED((NPAD, DH), jnp.float32),
            pltpu.VMEM_SHARED((NPAD, 16), jnp.float32),
            pltpu.SemaphoreType.DMA,
        ],
        compiler_params=pltpu.CompilerParams(use_tc_tiling_on_sc=False),
    )(xl, xr, src3, dst3, z64, z16, o16)


BR = 1000


def _tc_body(x_ref, agg_ref, cnt_ref, w1_ref, b1_ref, w2_ref, b2_ref,
             out_ref, acc_ref):
    i = pl.program_id(0)
    dot = lambda a, b: lax.dot_general(a, b, (((1,), (0,)), ((), ())),
                                       preferred_element_type=jnp.float32,
                                       precision=lax.Precision.HIGHEST)
    y = (dot(x_ref[...], w1_ref[...])
         + dot(agg_ref[0], w1_ref[0:DH, :])
         + dot(agg_ref[1], w1_ref[DH:D, :]))
    h = jnp.maximum(y + b1_ref[...], 0.0)
    w_col = 1.0 + cnt_ref[0, :, 0:1] + cnt_ref[1, :, 0:1]
    p = jnp.sum(h * w_col, axis=0, keepdims=True)

    @pl.when(i == 0)
    def _():
        acc_ref[...] = p

    @pl.when(i > 0)
    def _():
        acc_ref[...] = acc_ref[...] + p

    @pl.when(i == (N // BR) - 1)
    def _():
        out_ref[...] = dot(acc_ref[...], w2_ref[...]) + float(N) * b2_ref[...]


@jax.jit
def _tc_stage(x, aggp, cntp, W1, b1, W2, b2):
    return pl.pallas_call(
        _tc_body,
        grid=(N // BR,),
        in_specs=[
            pl.BlockSpec((BR, D), lambda i: (i, 0)),
            pl.BlockSpec((NC, BR, DH), lambda i: (0, i, 0)),
            pl.BlockSpec((NC, BR, 16), lambda i: (0, i, 0)),
            pl.BlockSpec((D, C), lambda i: (0, 0)),
            pl.BlockSpec((1, C), lambda i: (0, 0)),
            pl.BlockSpec((C, C), lambda i: (0, 0)),
            pl.BlockSpec((1, C), lambda i: (0, 0)),
        ],
        out_specs=pl.BlockSpec((1, C), lambda i: (0, 0)),
        out_shape=jax.ShapeDtypeStruct((1, C), jnp.float32),
        scratch_shapes=[pltpu.VMEM((1, C), jnp.float32)],
    )(x, aggp, cntp, W1, b1, W2, b2)


def kernel(x, edge_index, W1, b1, W2, b2):
    xl = x[:, :DH]
    xr = x[:, DH:]
    src3 = edge_index[0].reshape(NS, NCH, K)
    dst3 = edge_index[1].reshape(NS, NCH, K)
    z64 = jnp.zeros((RPT, DH), jnp.float32)
    z16 = jnp.zeros((RPT, 16), jnp.float32)
    o16 = jnp.ones((K, 16), jnp.float32)
    aggp, cntp = _sc_scatter(xl, xr, src3, dst3, z64, z16, o16)
    return _tc_stage(x, aggp, cntp, W1, b1.reshape(1, C), W2, b2.reshape(1, C))


# speedup vs baseline: 15.5179x; 1.6645x over previous
import jax
import jax.numpy as jnp
from jax import lax
from jax.experimental import pallas as pl
from jax.experimental.pallas import tpu as pltpu
from jax.experimental.pallas import tpu_sc as plsc

N, E, D, C = 10000, 320000, 128, 128
NC, NS = 2, 16
DH = D // NC
EPW = E // NS
K = 125
NCH = EPW // K
CPC = NCH // NC
NPAD = 10240
RPT = NPAD // NS
NBUF = 4


def _sc_body(xl_hbm, xr_hbm, src_hbm, dst_hbm, z64_hbm, z16_hbm, o16_hbm,
             agg_out, cnt_out,
             src_v, dst_v, rows_v, ones_v, acc_sh, cnt_sh, sems):
    cid = lax.axis_index("c")
    sid = lax.axis_index("s")

    row0 = sid * RPT
    pltpu.sync_copy(z64_hbm, acc_sh.at[pl.ds(row0, RPT)])
    pltpu.sync_copy(z16_hbm, cnt_sh.at[pl.ds(row0, RPT)])

    pltpu.sync_copy(src_hbm.at[sid], src_v)
    pltpu.sync_copy(dst_hbm.at[sid], dst_v)
    pltpu.sync_copy(o16_hbm, ones_v)

    plsc.subcore_barrier()

    def gather(j, b):
        @pl.when(cid == 0)
        def _():
            pltpu.async_copy(xl_hbm.at[src_v.at[j]], rows_v.at[b], sems.at[b])

        @pl.when(cid == 1)
        def _():
            pltpu.async_copy(xr_hbm.at[src_v.at[j]], rows_v.at[b], sems.at[b])

    for b in range(NBUF):
        gather(b, b)

    def step(t, carry):
        jj = t * NBUF
        for b in range(NBUF):
            j = jj + b
            pltpu.make_async_copy(xl_hbm.at[src_v.at[j]], rows_v.at[b],
                                  sems.at[b]).wait()
            pltpu.sync_copy(rows_v.at[b], acc_sh.at[dst_v.at[j]], add=True)

            @pl.when((j >= cid * CPC) & (j < (cid + 1) * CPC))
            def _():
                pltpu.sync_copy(ones_v, cnt_sh.at[src_v.at[j]], add=True)

            @pl.when(j + NBUF < NCH)
            def _():
                gather(j + NBUF, b)
        return carry

    lax.fori_loop(0, NCH // NBUF, step, 0)

    plsc.subcore_barrier()

    pltpu.sync_copy(acc_sh.at[pl.ds(row0, RPT)],
                    agg_out.at[cid, pl.ds(row0, RPT)])
    pltpu.sync_copy(cnt_sh.at[pl.ds(row0, RPT)],
                    cnt_out.at[cid, pl.ds(row0, RPT)])


@jax.jit
def _sc_scatter(xl, xr, src3, dst3, z64, z16, o16):
    mesh = plsc.VectorSubcoreMesh(core_axis_name="c", subcore_axis_name="s")
    return pl.kernel(
        _sc_body,
        out_type=(jax.ShapeDtypeStruct((NC, NPAD, DH), jnp.float32),
                  jax.ShapeDtypeStruct((NC, NPAD, 16), jnp.float32)),
        mesh=mesh,
        scratch_types=[
            pltpu.VMEM((NCH, K), jnp.int32),
            pltpu.VMEM((NCH, K), jnp.int32),
            pltpu.VMEM((NBUF, K, DH), jnp.float32),
            pltpu.VMEM((K, 16), jnp.float32),
            pltpu.VMEM_SHARED((NPAD, DH), jnp.float32),
            pltpu.VMEM_SHARED((NPAD, 16), jnp.float32),
            pltpu.SemaphoreType.DMA((NBUF,)),
        ],
        compiler_params=pltpu.CompilerParams(use_tc_tiling_on_sc=False),
    )(xl, xr, src3, dst3, z64, z16, o16)


BR = 1000


def _tc_body(x_ref, agg_ref, cnt_ref, w1_ref, b1_ref, w2_ref, b2_ref,
             out_ref, acc_ref):
    i = pl.program_id(0)
    dot = lambda a, b: lax.dot_general(a, b, (((1,), (0,)), ((), ())),
                                       preferred_element_type=jnp.float32,
                                       precision=lax.Precision.HIGHEST)
    y = (dot(x_ref[...], w1_ref[...])
         + dot(agg_ref[0], w1_ref[0:DH, :])
         + dot(agg_ref[1], w1_ref[DH:D, :]))
    h = jnp.maximum(y + b1_ref[...], 0.0)
    w_col = 1.0 + cnt_ref[0, :, 0:1] + cnt_ref[1, :, 0:1]
    p = jnp.sum(h * w_col, axis=0, keepdims=True)

    @pl.when(i == 0)
    def _():
        acc_ref[...] = p

    @pl.when(i > 0)
    def _():
        acc_ref[...] = acc_ref[...] + p

    @pl.when(i == (N // BR) - 1)
    def _():
        out_ref[...] = dot(acc_ref[...], w2_ref[...]) + float(N) * b2_ref[...]


@jax.jit
def _tc_stage(x, aggp, cntp, W1, b1, W2, b2):
    return pl.pallas_call(
        _tc_body,
        grid=(N // BR,),
        in_specs=[
            pl.BlockSpec((BR, D), lambda i: (i, 0)),
            pl.BlockSpec((NC, BR, DH), lambda i: (0, i, 0)),
            pl.BlockSpec((NC, BR, 16), lambda i: (0, i, 0)),
            pl.BlockSpec((D, C), lambda i: (0, 0)),
            pl.BlockSpec((1, C), lambda i: (0, 0)),
            pl.BlockSpec((C, C), lambda i: (0, 0)),
            pl.BlockSpec((1, C), lambda i: (0, 0)),
        ],
        out_specs=pl.BlockSpec((1, C), lambda i: (0, 0)),
        out_shape=jax.ShapeDtypeStruct((1, C), jnp.float32),
        scratch_shapes=[pltpu.VMEM((1, C), jnp.float32)],
    )(x, aggp, cntp, W1, b1, W2, b2)


def kernel(x, edge_index, W1, b1, W2, b2):
    xl = x[:, :DH]
    xr = x[:, DH:]
    src3 = edge_index[0].reshape(NS, NCH, K)
    dst3 = edge_index[1].reshape(NS, NCH, K)
    z64 = jnp.zeros((RPT, DH), jnp.float32)
    z16 = jnp.zeros((RPT, 16), jnp.float32)
    o16 = jnp.ones((K, 16), jnp.float32)
    aggp, cntp = _sc_scatter(xl, xr, src3, dst3, z64, z16, o16)
    return _tc_stage(x, aggp, cntp, W1, b1.reshape(1, C), W2, b2.reshape(1, C))


# speedup vs baseline: 17.0458x; 1.0985x over previous
import jax
import jax.numpy as jnp
from jax import lax
from jax.experimental import pallas as pl
from jax.experimental.pallas import tpu as pltpu
from jax.experimental.pallas import tpu_sc as plsc

N, E, D, C = 10000, 320000, 128, 128
NC, NS = 2, 16
DH = D // NC
EPW = E // NS
K = 125
NCH = EPW // K
CPC = NCH // NC
NPAD = 10240
RPT = NPAD // NS
NBUF = 4


def _sc_body(xl_hbm, xr_hbm, src_hbm, dst_hbm, z64_hbm, z16_hbm, o16_hbm,
             agg_out, cnt_out,
             src_v, dst_v, rows_v, ones_v, acc_sh, cnt_sh, sems, osem):
    cid = lax.axis_index("c")
    sid = lax.axis_index("s")

    row0 = sid * RPT
    pltpu.sync_copy(z64_hbm, acc_sh.at[pl.ds(row0, RPT)])
    pltpu.sync_copy(z16_hbm, cnt_sh.at[pl.ds(row0, RPT)])

    pltpu.sync_copy(src_hbm.at[sid], src_v)
    pltpu.sync_copy(dst_hbm.at[sid], dst_v)
    pltpu.sync_copy(o16_hbm, ones_v)

    plsc.subcore_barrier()

    def gather(j, b):
        @pl.when(cid == 0)
        def _():
            pltpu.async_copy(xl_hbm.at[src_v.at[j]], rows_v.at[b], sems.at[b])

        @pl.when(cid == 1)
        def _():
            pltpu.async_copy(xr_hbm.at[src_v.at[j]], rows_v.at[b], sems.at[b])

    for b in range(NBUF):
        gather(b, b)

    def step(t, carry):
        jj = t * NBUF
        for b in range(NBUF):
            j = jj + b
            pltpu.make_async_copy(xl_hbm.at[src_v.at[j]], rows_v.at[b],
                                  sems.at[b]).wait()
            pltpu.sync_copy(rows_v.at[b], acc_sh.at[dst_v.at[j]], add=True)

            @pl.when((j >= cid * CPC) & (j < (cid + 1) * CPC))
            def _():
                pltpu.async_copy(ones_v, cnt_sh.at[src_v.at[j]], osem,
                                 add=True)

            @pl.when(j + NBUF < NCH)
            def _():
                gather(j + NBUF, b)
        return carry

    lax.fori_loop(0, NCH // NBUF, step, 0)

    def drain(t, carry):
        pltpu.make_async_copy(ones_v, cnt_sh.at[src_v.at[0]], osem).wait()
        return carry

    lax.fori_loop(0, CPC, drain, 0)

    plsc.subcore_barrier()

    pltpu.sync_copy(acc_sh.at[pl.ds(row0, RPT)],
                    agg_out.at[cid, pl.ds(row0, RPT)])
    pltpu.sync_copy(cnt_sh.at[pl.ds(row0, RPT)],
                    cnt_out.at[cid, pl.ds(row0, RPT)])


@jax.jit
def _sc_scatter(xl, xr, src3, dst3, z64, z16, o16):
    mesh = plsc.VectorSubcoreMesh(core_axis_name="c", subcore_axis_name="s")
    return pl.kernel(
        _sc_body,
        out_type=(jax.ShapeDtypeStruct((NC, NPAD, DH), jnp.float32),
                  jax.ShapeDtypeStruct((NC, NPAD, 16), jnp.float32)),
        mesh=mesh,
        scratch_types=[
            pltpu.VMEM((NCH, K), jnp.int32),
            pltpu.VMEM((NCH, K), jnp.int32),
            pltpu.VMEM((NBUF, K, DH), jnp.float32),
            pltpu.VMEM((K, 16), jnp.float32),
            pltpu.VMEM_SHARED((NPAD, DH), jnp.float32),
            pltpu.VMEM_SHARED((NPAD, 16), jnp.float32),
            pltpu.SemaphoreType.DMA((NBUF,)),
            pltpu.SemaphoreType.DMA,
        ],
        compiler_params=pltpu.CompilerParams(use_tc_tiling_on_sc=False),
    )(xl, xr, src3, dst3, z64, z16, o16)


BR = 2000


def _tc_body(x_ref, agg_ref, cnt_ref, w1_ref, b1_ref, w2_ref, b2_ref,
             out_ref, acc_ref):
    i = pl.program_id(0)
    def dot(a, b, prec):
        return lax.dot_general(a, b, (((1,), (0,)), ((), ())),
                               preferred_element_type=jnp.float32,
                               precision=prec)
    hp = lax.Precision.HIGHEST
    y = (dot(x_ref[...], w1_ref[...], None)
         + dot(agg_ref[0], w1_ref[0:DH, :], None)
         + dot(agg_ref[1], w1_ref[DH:D, :], None))
    h = jnp.maximum(y + b1_ref[...], 0.0)
    w_col = 1.0 + cnt_ref[0, :, 0:1] + cnt_ref[1, :, 0:1]
    p = jnp.sum(h * w_col, axis=0, keepdims=True)

    @pl.when(i == 0)
    def _():
        acc_ref[...] = p

    @pl.when(i > 0)
    def _():
        acc_ref[...] = acc_ref[...] + p

    @pl.when(i == (N // BR) - 1)
    def _():
        out_ref[...] = (dot(acc_ref[...], w2_ref[...], hp)
                        + float(N) * b2_ref[...])


@jax.jit
def _tc_stage(x, aggp, cntp, W1, b1, W2, b2):
    return pl.pallas_call(
        _tc_body,
        grid=(N // BR,),
        in_specs=[
            pl.BlockSpec((BR, D), lambda i: (i, 0)),
            pl.BlockSpec((NC, BR, DH), lambda i: (0, i, 0)),
            pl.BlockSpec((NC, BR, 16), lambda i: (0, i, 0)),
            pl.BlockSpec((D, C), lambda i: (0, 0)),
            pl.BlockSpec((1, C), lambda i: (0, 0)),
            pl.BlockSpec((C, C), lambda i: (0, 0)),
            pl.BlockSpec((1, C), lambda i: (0, 0)),
        ],
        out_specs=pl.BlockSpec((1, C), lambda i: (0, 0)),
        out_shape=jax.ShapeDtypeStruct((1, C), jnp.float32),
        scratch_shapes=[pltpu.VMEM((1, C), jnp.float32)],
    )(x, aggp, cntp, W1, b1, W2, b2)


def kernel(x, edge_index, W1, b1, W2, b2):
    xl = x[:, :DH]
    xr = x[:, DH:]
    src3 = edge_index[0].reshape(NS, NCH, K)
    dst3 = edge_index[1].reshape(NS, NCH, K)
    z64 = jnp.zeros((RPT, DH), jnp.float32)
    z16 = jnp.zeros((RPT, 16), jnp.float32)
    o16 = jnp.ones((K, 16), jnp.float32)
    aggp, cntp = _sc_scatter(xl, xr, src3, dst3, z64, z16, o16)
    return _tc_stage(x, aggp, cntp, W1, b1.reshape(1, C), W2, b2.reshape(1, C))


# speedup vs baseline: 18.7156x; 1.0980x over previous
import jax
import jax.numpy as jnp
from jax import lax
from jax.experimental import pallas as pl
from jax.experimental.pallas import tpu as pltpu
from jax.experimental.pallas import tpu_sc as plsc

N, E, D, C = 10000, 320000, 128, 128
NC, NS = 2, 16
DH = D // NC
EPW = E // NS
K = 125
NCH = EPW // K
CPC = NCH // NC
NPAD = 10240
RPT = NPAD // NS
NBUF = 4
ZR = 128


def _sc_body(xl_hbm, xr_hbm, idx_hbm, z64_hbm, z16_hbm, o16_hbm,
             agg_out, cnt_out,
             src_v, dst_v, rows_v, ones_v, acc_sh, cnt_sh, sems, osem):
    cid = lax.axis_index("c")
    sid = lax.axis_index("s")

    row0 = sid * RPT
    pltpu.sync_copy(z64_hbm, acc_sh.at[pl.ds(row0, RPT)])
    pltpu.sync_copy(z16_hbm, cnt_sh.at[pl.ds(row0, RPT)])

    pltpu.sync_copy(idx_hbm.at[0, sid], src_v)
    pltpu.sync_copy(idx_hbm.at[1, sid], dst_v)
    pltpu.sync_copy(o16_hbm, ones_v)

    plsc.subcore_barrier()

    def gather(j, b):
        @pl.when(cid == 0)
        def _():
            pltpu.async_copy(xl_hbm.at[src_v.at[j]], rows_v.at[b], sems.at[b])

        @pl.when(cid == 1)
        def _():
            pltpu.async_copy(xr_hbm.at[src_v.at[j]], rows_v.at[b], sems.at[b])

    for b in range(NBUF):
        gather(b, b)

    def step(t, carry):
        jj = t * NBUF
        for b in range(NBUF):
            j = jj + b
            pltpu.make_async_copy(xl_hbm.at[src_v.at[j]], rows_v.at[b],
                                  sems.at[b]).wait()
            pltpu.sync_copy(rows_v.at[b], acc_sh.at[dst_v.at[j]], add=True)

            @pl.when((j >= cid * CPC) & (j < (cid + 1) * CPC))
            def _():
                pltpu.async_copy(ones_v, cnt_sh.at[src_v.at[j]], osem,
                                 add=True)

            @pl.when(j + NBUF < NCH)
            def _():
                gather(j + NBUF, b)
        return carry

    lax.fori_loop(0, NCH // NBUF, step, 0)

    def drain(t, carry):
        pltpu.make_async_copy(ones_v, cnt_sh.at[src_v.at[0]], osem).wait()
        return carry

    lax.fori_loop(0, CPC, drain, 0)

    plsc.subcore_barrier()

    pltpu.sync_copy(acc_sh.at[pl.ds(row0, RPT)],
                    agg_out.at[cid, pl.ds(row0, RPT)])
    pltpu.sync_copy(cnt_sh.at[pl.ds(row0, RPT)],
                    cnt_out.at[cid, pl.ds(row0, RPT)])


@jax.jit
def _sc_scatter(xl, xr, idx4, z64, z16, o16):
    mesh = plsc.VectorSubcoreMesh(core_axis_name="c", subcore_axis_name="s")
    return pl.kernel(
        _sc_body,
        out_type=(jax.ShapeDtypeStruct((NC, NPAD, DH), jnp.float32),
                  jax.ShapeDtypeStruct((NC, NPAD, 16), jnp.float32)),
        mesh=mesh,
        scratch_types=[
            pltpu.VMEM((NCH, K), jnp.int32),
            pltpu.VMEM((NCH, K), jnp.int32),
            pltpu.VMEM((NBUF, K, DH), jnp.float32),
            pltpu.VMEM((K, 16), jnp.float32),
            pltpu.VMEM_SHARED((NPAD, DH), jnp.float32),
            pltpu.VMEM_SHARED((NPAD, 16), jnp.float32),
            pltpu.SemaphoreType.DMA((NBUF,)),
            pltpu.SemaphoreType.DMA,
        ],
        compiler_params=pltpu.CompilerParams(use_tc_tiling_on_sc=False),
    )(xl, xr, idx4, z64, z16, o16)


BR = 2000


def _tc_body(x_ref, agg_ref, cnt_ref, w1_ref, b1_ref, w2_ref, b2_ref,
             out_ref, acc_ref):
    i = pl.program_id(0)
    def dot(a, b, prec):
        return lax.dot_general(a, b, (((1,), (0,)), ((), ())),
                               preferred_element_type=jnp.float32,
                               precision=prec)
    hp = lax.Precision.HIGHEST
    y = (dot(x_ref[...], w1_ref[...], None)
         + dot(agg_ref[0], w1_ref[0:DH, :], None)
         + dot(agg_ref[1], w1_ref[DH:D, :], None))
    h = jnp.maximum(y + b1_ref[...], 0.0)
    w_col = 1.0 + cnt_ref[0, :, 0:1] + cnt_ref[1, :, 0:1]
    p = jnp.sum(h * w_col, axis=0, keepdims=True)

    @pl.when(i == 0)
    def _():
        acc_ref[...] = p

    @pl.when(i > 0)
    def _():
        acc_ref[...] = acc_ref[...] + p

    @pl.when(i == (N // BR) - 1)
    def _():
        out_ref[...] = (dot(acc_ref[...], w2_ref[...], hp)
                        + float(N) * b2_ref[...])


@jax.jit
def _tc_stage(x, aggp, cntp, W1, b1, W2, b2):
    return pl.pallas_call(
        _tc_body,
        grid=(N // BR,),
        in_specs=[
            pl.BlockSpec((BR, D), lambda i: (i, 0)),
            pl.BlockSpec((NC, BR, DH), lambda i: (0, i, 0)),
            pl.BlockSpec((NC, BR, 16), lambda i: (0, i, 0)),
            pl.BlockSpec((D, C), lambda i: (0, 0)),
            pl.BlockSpec((1, C), lambda i: (0, 0)),
            pl.BlockSpec((C, C), lambda i: (0, 0)),
            pl.BlockSpec((1, C), lambda i: (0, 0)),
        ],
        out_specs=pl.BlockSpec((1, C), lambda i: (0, 0)),
        out_shape=jax.ShapeDtypeStruct((1, C), jnp.float32),
        scratch_shapes=[pltpu.VMEM((1, C), jnp.float32)],
    )(x, aggp, cntp, W1, b1, W2, b2)


def kernel(x, edge_index, W1, b1, W2, b2):
    xl = x[:, :DH]
    xr = x[:, DH:]
    idx4 = edge_index.reshape(2, NS, NCH, K)
    z64 = jnp.zeros((RPT, DH), jnp.float32)
    z16 = jnp.zeros((RPT, 16), jnp.float32)
    o16 = jnp.ones((K, 16), jnp.float32)
    aggp, cntp = _sc_scatter(xl, xr, idx4, z64, z16, o16)
    return _tc_stage(x, aggp, cntp, W1, b1.reshape(1, C), W2, b2.reshape(1, C))
